# Initial kernel scaffold; baseline (speedup 1.0000x reference)
import jax
import jax.numpy as jnp
from jax.experimental import pallas as pl


def kernel(x, embedding, W1, b1, wg1, bg1, W2, b2, wg2, bg2, Wi, Wh, bi, bh, edge_index, edge_attr, entA_pos, entB_pos):
    raise NotImplementedError("write your pallas kernel here")


# speedup vs baseline: 1.0377x; 1.0377x over previous
import functools

import jax
import jax.numpy as jnp
from jax.experimental import pallas as pl
from jax.experimental.pallas import tpu as pltpu

B, L, D, H, NUM_LABELS = 64, 512, 128, 128, 46
N = B * L
E = N * 12


def _lstm_pool_body(posA_ref, posB_ref, emb_ref, wih_ref, bias_ref,
                    accA_ref, accB_ref, h_ref, c_ref):
    t = pl.program_id(0)

    @pl.when(t == 0)
    def _init():
        h_ref[...] = jnp.zeros_like(h_ref)
        c_ref[...] = jnp.zeros_like(c_ref)
        accA_ref[...] = jnp.zeros_like(accA_ref)
        accB_ref[...] = jnp.zeros_like(accB_ref)

    x_t = emb_ref[0]
    xh = jnp.concatenate([x_t, h_ref[...]], axis=1)
    gates = jnp.dot(xh, wih_ref[...],
                    preferred_element_type=jnp.float32,
                    precision=jax.lax.Precision.HIGHEST) + bias_ref[...]
    i = jax.nn.sigmoid(gates[:, 0:H])
    f = jax.nn.sigmoid(gates[:, H:2 * H])
    g = jnp.tanh(gates[:, 2 * H:3 * H])
    o = jax.nn.sigmoid(gates[:, 3 * H:4 * H])
    c = f * c_ref[...] + i * g
    hn = o * jnp.tanh(c)
    h_ref[...] = hn
    c_ref[...] = c

    maskA = jnp.sum((posA_ref[...] == t).astype(jnp.float32), axis=1) * 0.25
    maskB = jnp.sum((posB_ref[...] == t).astype(jnp.float32), axis=1) * 0.25
    accA_ref[...] += maskA[:, None] * hn
    accB_ref[...] += maskB[:, None] * hn


def _lstm_pool(embedding, Wi, Wh, bi, bh, entA_pos, entB_pos):
    wih = jnp.concatenate([Wi, Wh], axis=0)
    bias = (bi + bh).reshape(1, 4 * H)
    grid = (L,)
    return pl.pallas_call(
        _lstm_pool_body,
        grid=grid,
        in_specs=[
            pl.BlockSpec((B, 4), lambda t: (0, 0)),
            pl.BlockSpec((B, 4), lambda t: (0, 0)),
            pl.BlockSpec((1, B, D), lambda t: (t, 0, 0)),
            pl.BlockSpec((D + H, 4 * H), lambda t: (0, 0)),
            pl.BlockSpec((1, 4 * H), lambda t: (0, 0)),
        ],
        out_specs=[
            pl.BlockSpec((B, H), lambda t: (0, 0)),
            pl.BlockSpec((B, H), lambda t: (0, 0)),
        ],
        out_shape=[
            jax.ShapeDtypeStruct((B, H), jnp.float32),
            jax.ShapeDtypeStruct((B, H), jnp.float32),
        ],
        scratch_shapes=[
            pltpu.VMEM((B, H), jnp.float32),
            pltpu.VMEM((B, H), jnp.float32),
        ],
    )(entA_pos, entB_pos, jnp.swapaxes(embedding, 0, 1), wih, bias)


def _sgcn_layer(x, src, dst, lab, W, b, wg, bg):
    h = x @ W
    msg = h[src] + b[lab]
    gate = jax.nn.sigmoid((x @ wg)[src] + bg[lab])
    agg = jnp.zeros((N, D), jnp.float32).at[dst].add(gate[:, None] * msg)
    return jax.nn.relu(agg)


def kernel(x, embedding, W1, b1, wg1, bg1, W2, b2, wg2, bg2,
           Wi, Wh, bi, bh, edge_index, edge_attr, entA_pos, entB_pos):
    src = edge_index[0]
    dst = edge_index[1]
    h = _sgcn_layer(x, src, dst, edge_attr, W1, b1, wg1, bg1)
    h = _sgcn_layer(h, src, dst, edge_attr, W2, b2, wg2, bg2)
    node_hidden = h.reshape(B, L, D)

    wordA, wordB = _lstm_pool(embedding, Wi, Wh, bi, bh, entA_pos, entB_pos)

    nodeA = jnp.take_along_axis(node_hidden, entA_pos[:, :, None], axis=1).mean(axis=1)
    nodeB = jnp.take_along_axis(node_hidden, entB_pos[:, :, None], axis=1).mean(axis=1)
    return (nodeA.reshape(-1), nodeB.reshape(-1),
            wordA.reshape(-1), wordB.reshape(-1))


# speedup vs baseline: 1.8445x; 1.7774x over previous
import functools

import jax
import jax.numpy as jnp
from jax import lax
from jax.experimental import pallas as pl
from jax.experimental.pallas import tpu as pltpu
from jax.experimental.pallas import tpu_sc as plsc

B, L, D, H, NUM_LABELS = 64, 512, 128, 128, 46
N = B * L
E = N * 12


def _lstm_pool_body(posA_ref, posB_ref, emb_ref, wih_ref, bias_ref,
                    accA_ref, accB_ref, h_ref, c_ref):
    t = pl.program_id(0)

    @pl.when(t == 0)
    def _init():
        h_ref[...] = jnp.zeros_like(h_ref)
        c_ref[...] = jnp.zeros_like(c_ref)
        accA_ref[...] = jnp.zeros_like(accA_ref)
        accB_ref[...] = jnp.zeros_like(accB_ref)

    x_t = emb_ref[0]
    xh = jnp.concatenate([x_t, h_ref[...]], axis=1)
    gates = jnp.dot(xh, wih_ref[...],
                    preferred_element_type=jnp.float32,
                    precision=jax.lax.Precision.HIGHEST) + bias_ref[...]
    i = jax.nn.sigmoid(gates[:, 0:H])
    f = jax.nn.sigmoid(gates[:, H:2 * H])
    g = jnp.tanh(gates[:, 2 * H:3 * H])
    o = jax.nn.sigmoid(gates[:, 3 * H:4 * H])
    c = f * c_ref[...] + i * g
    hn = o * jnp.tanh(c)
    h_ref[...] = hn
    c_ref[...] = c

    maskA = jnp.sum((posA_ref[...] == t).astype(jnp.float32), axis=1) * 0.25
    maskB = jnp.sum((posB_ref[...] == t).astype(jnp.float32), axis=1) * 0.25
    accA_ref[...] += maskA[:, None] * hn
    accB_ref[...] += maskB[:, None] * hn


def _lstm_pool(embedding, Wi, Wh, bi, bh, entA_pos, entB_pos):
    wih = jnp.concatenate([Wi, Wh], axis=0)
    bias = (bi + bh).reshape(1, 4 * H)
    return pl.pallas_call(
        _lstm_pool_body,
        grid=(L,),
        in_specs=[
            pl.BlockSpec((B, 4), lambda t: (0, 0)),
            pl.BlockSpec((B, 4), lambda t: (0, 0)),
            pl.BlockSpec((1, B, D), lambda t: (t, 0, 0)),
            pl.BlockSpec((D + H, 4 * H), lambda t: (0, 0)),
            pl.BlockSpec((1, 4 * H), lambda t: (0, 0)),
        ],
        out_specs=[
            pl.BlockSpec((B, H), lambda t: (0, 0)),
            pl.BlockSpec((B, H), lambda t: (0, 0)),
        ],
        out_shape=[
            jax.ShapeDtypeStruct((B, H), jnp.float32),
            jax.ShapeDtypeStruct((B, H), jnp.float32),
        ],
        scratch_shapes=[
            pltpu.VMEM((B, H), jnp.float32),
            pltpu.VMEM((B, H), jnp.float32),
        ],
    )(entA_pos, entB_pos, jnp.swapaxes(embedding, 0, 1), wih, bias)


NW = 32
EW = E // NW
CH = 1024
NCH = EW // CH
G = 64
CAP = EW + G
NSLOT = 512
DUMP = NSLOT
ACC_ROWS = 640
RPW = ACC_ROWS // 16


def _sgcn2_body(ei, labh, map2h, g2linh, h2hath, b2h, bg2h,
                outh,
                nval, gval, cs_src, cs_meta,
                cb_s, cb_d, cb_l, rows, msg, b2v, bg2v, gateb, sidx,
                acc):
    c = lax.axis_index("c")
    s = lax.axis_index("s")
    wid = c * 16 + s
    base = wid * EW
    i16 = lax.iota(jnp.int32, 16)
    zero16 = jnp.zeros((16,), jnp.int32)

    for r in range(RPW):
        for k in range(8):
            msg[r, pl.ds(k * 16, 16)] = jnp.zeros((16,), jnp.float32)
    pltpu.sync_copy(msg.at[pl.ds(0, RPW)], acc.at[pl.ds(s * RPW, RPW)])

    pltpu.sync_copy(map2h, nval)
    pltpu.sync_copy(g2linh, gval)
    pltpu.sync_copy(b2h, b2v)
    pltpu.sync_copy(bg2h, bg2v)

    cnt = jnp.int32(0)
    for ch in range(NCH):
        pltpu.sync_copy(ei.at[0, pl.ds(base + ch * CH, CH)], cb_s)
        pltpu.sync_copy(ei.at[1, pl.ds(base + ch * CH, CH)], cb_d)
        pltpu.sync_copy(labh.at[pl.ds(base + ch * CH, CH)], cb_l)

        def fbody(i, cnt):
            dv = cb_d[pl.ds(i * 16, 16)]
            sv = cb_s[pl.ds(i * 16, 16)]
            lv = cb_l[pl.ds(i * 16, 16)]
            slot = plsc.load_gather(nval, [dv])
            m = slot >= 0
            mi = m.astype(jnp.int32)
            pos = cnt + lax.cumsum(mi) - 1
            plsc.store_scatter(cs_src, [pos], sv, mask=m)
            plsc.store_scatter(cs_meta, [pos], slot * 64 + lv, mask=m)
            return cnt + jnp.sum(mi)

        cnt = lax.fori_loop(0, CH // 16, fbody, cnt)

    n2 = cnt
    npad = ((n2 + G - 1) // G) * G

    def pbody(i, _):
        pos = n2 + i * 16 + i16
        m = pos < npad
        plsc.store_scatter(cs_src, [pos], zero16, mask=m)
        plsc.store_scatter(cs_meta, [pos], zero16 + DUMP * 64, mask=m)
        return 0

    lax.fori_loop(0, G // 16, pbody, 0)
    plsc.subcore_barrier()

    def gbody(g, _):
        pltpu.sync_copy(h2hath.at[cs_src.at[pl.ds(g * G, G)]], rows)
        for q in range(G // 16):
            srcv = cs_src[pl.ds(g * G + q * 16, 16)]
            metav = cs_meta[pl.ds(g * G + q * 16, 16)]
            labv = jnp.bitwise_and(metav, 63)
            gl = plsc.load_gather(gval, [srcv])
            bg = plsc.load_gather(bg2v, [labv])
            gate = 1.0 / (1.0 + jnp.exp(-(gl + bg)))
            gateb[pl.ds(q * 16, 16)] = gate
            sidx[pl.ds(q * 16, 16)] = lax.shift_right_logical(metav, 6)

        def ebody(j, _):
            gsp = plsc.load_gather(gateb, [zero16 + j])
            lsp = jnp.bitwise_and(
                plsc.load_gather(cs_meta, [zero16 + (g * G + j)]), 63)
            jsp = zero16 + j
            for k in range(8):
                col = k * 16 + i16
                b2r = plsc.load_gather(b2v, [lsp * 128 + col])
                rowk = plsc.load_gather(rows, [jsp, col])
                plsc.store_scatter(msg, [jsp, col], gsp * (rowk + b2r))
            return 0

        lax.fori_loop(0, G, ebody, 0)
        pltpu.sync_copy(msg, acc.at[sidx], add=True)
        return 0

    lax.fori_loop(0, npad // G, gbody, 0)
    plsc.subcore_barrier()

    pltpu.sync_copy(acc.at[pl.ds(s * RPW, RPW)], msg.at[pl.ds(0, RPW)])
    pltpu.sync_copy(msg.at[pl.ds(0, RPW)], outh.at[c, pl.ds(s * RPW, RPW)])


def _sgcn2_sc(edge_index, edge_attr, map2, g2lin, h2hat, b2, bg2):
    mesh = plsc.VectorSubcoreMesh(core_axis_name="c", subcore_axis_name="s")
    bg2p = jnp.pad(bg2, (0, 64 - NUM_LABELS))
    k = pl.kernel(
        _sgcn2_body,
        out_type=jax.ShapeDtypeStruct((2, ACC_ROWS, D), jnp.float32),
        mesh=mesh,
        compiler_params=pltpu.CompilerParams(needs_layout_passes=False),
        scratch_types=[
            pltpu.VMEM((N,), jnp.int32),
            pltpu.VMEM((N,), jnp.float32),
            pltpu.VMEM((CAP,), jnp.int32),
            pltpu.VMEM((CAP,), jnp.int32),
            pltpu.VMEM((CH,), jnp.int32),
            pltpu.VMEM((CH,), jnp.int32),
            pltpu.VMEM((CH,), jnp.int32),
            pltpu.VMEM((G, D), jnp.float32),
            pltpu.VMEM((G, D), jnp.float32),
            pltpu.VMEM((NUM_LABELS * D,), jnp.float32),
            pltpu.VMEM((64,), jnp.float32),
            pltpu.VMEM((G,), jnp.float32),
            pltpu.VMEM((G,), jnp.int32),
            pltpu.VMEM_SHARED((ACC_ROWS, D), jnp.float32),
        ],
    )
    return k(edge_index, edge_attr, map2, g2lin, h2hat,
             b2.reshape(-1), bg2p)


def _sgcn_layer_dense(x, src, dst, lab, W, b, wg, bg):
    h = x @ W
    msg = h[src] + b[lab]
    gate = jax.nn.sigmoid((x @ wg)[src] + bg[lab])
    agg = jnp.zeros((N, D), jnp.float32).at[dst].add(gate[:, None] * msg)
    return jax.nn.relu(agg)


def kernel(x, embedding, W1, b1, wg1, bg1, W2, b2, wg2, bg2,
           Wi, Wh, bi, bh, edge_index, edge_attr, entA_pos, entB_pos):
    src = edge_index[0]
    dst = edge_index[1]

    h1 = _sgcn_layer_dense(x, src, dst, edge_attr, W1, b1, wg1, bg1)
    h2hat = h1 @ W2
    g2lin = h1 @ wg2

    idsA = (jnp.arange(B, dtype=jnp.int32)[:, None] * L + entA_pos).reshape(-1)
    idsB = (jnp.arange(B, dtype=jnp.int32)[:, None] * L + entB_pos).reshape(-1)
    needed = jnp.concatenate([idsA, idsB])
    map2 = jnp.full((N,), -1, jnp.int32).at[needed].set(
        jnp.arange(NSLOT, dtype=jnp.int32))

    acc2 = _sgcn2_sc(edge_index, edge_attr, map2, g2lin, h2hat, b2, bg2)
    h2 = jax.nn.relu(acc2[0, :NSLOT] + acc2[1, :NSLOT])

    slotsA = map2[idsA]
    slotsB = map2[idsB]
    nodeA = h2[slotsA].reshape(B, 4, D).mean(axis=1)
    nodeB = h2[slotsB].reshape(B, 4, D).mean(axis=1)

    wordA, wordB = _lstm_pool(embedding, Wi, Wh, bi, bh, entA_pos, entB_pos)

    return (nodeA.reshape(-1), nodeB.reshape(-1),
            wordA.reshape(-1), wordB.reshape(-1))


# speedup vs baseline: 23.1354x; 12.5426x over previous
import functools

import jax
import jax.numpy as jnp
from jax import lax
from jax.experimental import pallas as pl
from jax.experimental.pallas import tpu as pltpu
from jax.experimental.pallas import tpu_sc as plsc

B, L, D, H, NUM_LABELS = 64, 512, 128, 128, 46
N = B * L
E = N * 12


def _lstm_pool_body(posA_ref, posB_ref, emb_ref, wih_ref, bias_ref,
                    accA_ref, accB_ref, h_ref, c_ref):
    t = pl.program_id(0)

    @pl.when(t == 0)
    def _init():
        h_ref[...] = jnp.zeros_like(h_ref)
        c_ref[...] = jnp.zeros_like(c_ref)
        accA_ref[...] = jnp.zeros_like(accA_ref)
        accB_ref[...] = jnp.zeros_like(accB_ref)

    x_t = emb_ref[0]
    xh = jnp.concatenate([x_t, h_ref[...]], axis=1)
    gates = jnp.dot(xh, wih_ref[...],
                    preferred_element_type=jnp.float32,
                    precision=jax.lax.Precision.DEFAULT) + bias_ref[...]
    i = jax.nn.sigmoid(gates[:, 0:H])
    f = jax.nn.sigmoid(gates[:, H:2 * H])
    g = jnp.tanh(gates[:, 2 * H:3 * H])
    o = jax.nn.sigmoid(gates[:, 3 * H:4 * H])
    c = f * c_ref[...] + i * g
    hn = o * jnp.tanh(c)
    h_ref[...] = hn
    c_ref[...] = c

    maskA = jnp.sum((posA_ref[...] == t).astype(jnp.float32), axis=1) * 0.25
    maskB = jnp.sum((posB_ref[...] == t).astype(jnp.float32), axis=1) * 0.25
    accA_ref[...] += maskA[:, None] * hn
    accB_ref[...] += maskB[:, None] * hn


def _lstm_pool(embedding, Wi, Wh, bi, bh, entA_pos, entB_pos):
    wih = jnp.concatenate([Wi, Wh], axis=0)
    bias = (bi + bh).reshape(1, 4 * H)
    return pl.pallas_call(
        _lstm_pool_body,
        grid=(L,),
        in_specs=[
            pl.BlockSpec((B, 4), lambda t: (0, 0)),
            pl.BlockSpec((B, 4), lambda t: (0, 0)),
            pl.BlockSpec((1, B, D), lambda t: (t, 0, 0)),
            pl.BlockSpec((D + H, 4 * H), lambda t: (0, 0)),
            pl.BlockSpec((1, 4 * H), lambda t: (0, 0)),
        ],
        out_specs=[
            pl.BlockSpec((B, H), lambda t: (0, 0)),
            pl.BlockSpec((B, H), lambda t: (0, 0)),
        ],
        out_shape=[
            jax.ShapeDtypeStruct((B, H), jnp.float32),
            jax.ShapeDtypeStruct((B, H), jnp.float32),
        ],
        scratch_shapes=[
            pltpu.VMEM((B, H), jnp.float32),
            pltpu.VMEM((B, H), jnp.float32),
        ],
    )(entA_pos, entB_pos, jnp.swapaxes(embedding, 0, 1), wih, bias)


NW = 32
EW = E // NW
CH = 1024
NCH = EW // CH
G = 64
CAP = EW + G + 16
NSLOT = 512
DUMP2 = NSLOT
ACC2_ROWS = 640
RPW2 = ACC2_ROWS // 16
PS = 4096
DUMP1 = PS
ACC1_ROWS = 4224
GC = 64
DW = 2 * D
_SC_PARAMS = pltpu.CompilerParams(needs_layout_passes=False)
_MESH = dict(core_axis_name="c", subcore_axis_name="s")


def _sigmoid16(x):
    return 1.0 / (1.0 + jnp.exp(-x))


def _edge_scan_body(ei, labh, map2h,
                    csh, cmh, cnth, markh,
                    mapv, markv, cs_src, cs_meta, cb_s, cb_d, cb_l,
                    sidx, cntv, sem):
    c = lax.axis_index("c")
    s = lax.axis_index("s")
    wid = c * 16 + s
    base = wid * EW
    i16 = lax.iota(jnp.int32, 16)
    zero16 = jnp.zeros((16,), jnp.int32)
    one16 = zero16 + 1

    def _zb(i, _):
        markv[pl.ds(i * 16, 16)] = zero16
        return 0
    lax.fori_loop(0, N // 16, _zb, 0)

    pltpu.sync_copy(map2h, mapv)

    descs = [None, None]

    def _fire(ch, buf):
        return (
            pltpu.async_copy(ei.at[0, pl.ds(base + ch * CH, CH)],
                             cb_s.at[buf], sem),
            pltpu.async_copy(ei.at[1, pl.ds(base + ch * CH, CH)],
                             cb_d.at[buf], sem),
            pltpu.async_copy(labh.at[pl.ds(base + ch * CH, CH)],
                             cb_l.at[buf], sem),
        )

    descs[0] = _fire(0, 0)
    cnt = jnp.int32(0)
    for ch in range(NCH):
        cur = ch % 2
        for dsc in descs[cur]:
            dsc.wait()
        if ch + 1 < NCH:
            descs[1 - cur] = _fire(ch + 1, 1 - cur)

        def fbody(i, cnt):
            dv = cb_d[cur, pl.ds(i * 16, 16)]
            sv = cb_s[cur, pl.ds(i * 16, 16)]
            lv = cb_l[cur, pl.ds(i * 16, 16)]
            slot = plsc.load_gather(mapv, [dv])
            m = slot >= 0
            mi = m.astype(jnp.int32)
            pos = cnt + lax.cumsum(mi) - 1
            plsc.store_scatter(cs_src, [pos], sv, mask=m)
            plsc.store_scatter(cs_meta, [pos], slot * 64 + lv, mask=m)
            plsc.store_scatter(markv, [sv], one16, mask=m)
            return cnt + jnp.sum(mi)

        cnt = lax.fori_loop(0, CH // 16, fbody, cnt)

    n2 = cnt
    npad = ((n2 + G - 1) // G) * G

    def pbody(i, _):
        pos = n2 + i * 16 + i16
        m = pos < npad
        plsc.store_scatter(cs_src, [pos], zero16, mask=m)
        plsc.store_scatter(cs_meta, [pos], zero16 + DUMP2 * 64, mask=m)
        return 0

    lax.fori_loop(0, G // 16, pbody, 0)

    cntv[pl.ds(0, 16)] = zero16 + n2
    pltpu.sync_copy(cntv, cnth.at[wid])
    pltpu.sync_copy(markv, markh.at[wid])
    pltpu.sync_copy(cs_src, csh.at[pl.ds(wid * CAP, CAP)])
    pltpu.sync_copy(cs_meta, cmh.at[pl.ds(wid * CAP, CAP)])


def _edge_scan(edge_index, edge_attr, map2):
    k = pl.kernel(
        _edge_scan_body,
        out_type=(
            jax.ShapeDtypeStruct((NW * CAP,), jnp.int32),
            jax.ShapeDtypeStruct((NW * CAP,), jnp.int32),
            jax.ShapeDtypeStruct((NW, 16), jnp.int32),
            jax.ShapeDtypeStruct((NW, N), jnp.int32),
        ),
        mesh=plsc.VectorSubcoreMesh(**_MESH),
        compiler_params=_SC_PARAMS,
        scratch_types=[
            pltpu.VMEM((N,), jnp.int32),
            pltpu.VMEM((N,), jnp.int32),
            pltpu.VMEM((CAP,), jnp.int32),
            pltpu.VMEM((CAP,), jnp.int32),
            pltpu.VMEM((2, CH), jnp.int32),
            pltpu.VMEM((2, CH), jnp.int32),
            pltpu.VMEM((2, CH), jnp.int32),
            pltpu.VMEM((G,), jnp.int32),
            pltpu.VMEM((16,), jnp.int32),
            pltpu.SemaphoreType.DMA,
        ],
    )
    return k(edge_index, edge_attr, map2)


def _layer1_body(ei, labh, map1h, n1h, h1augh, b1h, bg1h,
                 h1ch,
                 mapv, cs_src, cs_meta, cb_s, cb_d, cb_l,
                 rows, msg, b1v, bg1v, gateb, sidx, n1b, sem,
                 acc):
    c = lax.axis_index("c")
    s = lax.axis_index("s")
    base = (c * 16 + s) * EW
    i16 = lax.iota(jnp.int32, 16)
    zero16 = jnp.zeros((16,), jnp.int32)

    pltpu.sync_copy(map1h, mapv)
    pltpu.sync_copy(b1h, b1v)
    pltpu.sync_copy(bg1h, bg1v)
    pltpu.sync_copy(n1h, n1b)
    n1 = jnp.max(n1b[pl.ds(0, 16)])
    npass = (n1 + PS - 1) // PS

    def passbody(p, _):
        lo = p * PS
        for r in range(GC):
            for k in range(8):
                msg[r, pl.ds(k * 16, 16)] = jnp.zeros((16,), jnp.float32)
        for q in range(5):
            rl = min(GC, ACC1_ROWS // 16 - q * GC)
            pltpu.sync_copy(
                msg.at[pl.ds(0, rl)],
                acc.at[pl.ds(s * (ACC1_ROWS // 16) + q * GC, rl)])
        plsc.subcore_barrier()

        descs = [None, None]

        def _fire(ch, buf):
            return (
                pltpu.async_copy(ei.at[0, pl.ds(base + ch * CH, CH)],
                                 cb_s.at[buf], sem),
                pltpu.async_copy(ei.at[1, pl.ds(base + ch * CH, CH)],
                                 cb_d.at[buf], sem),
                pltpu.async_copy(labh.at[pl.ds(base + ch * CH, CH)],
                                 cb_l.at[buf], sem),
            )

        descs[0] = _fire(0, 0)
        cnt = jnp.int32(0)
        for ch in range(NCH):
            cur = ch % 2
            for dsc in descs[cur]:
                dsc.wait()
            if ch + 1 < NCH:
                descs[1 - cur] = _fire(ch + 1, 1 - cur)

            def fbody(i, cnt):
                dv = cb_d[cur, pl.ds(i * 16, 16)]
                sv = cb_s[cur, pl.ds(i * 16, 16)]
                lv = cb_l[cur, pl.ds(i * 16, 16)]
                slot = plsc.load_gather(mapv, [dv])
                m = (slot >= lo) & (slot < lo + PS)
                mi = m.astype(jnp.int32)
                pos = cnt + lax.cumsum(mi) - 1
                plsc.store_scatter(cs_src, [pos], sv, mask=m)
                plsc.store_scatter(cs_meta, [pos], (slot - lo) * 64 + lv,
                                   mask=m)
                return cnt + jnp.sum(mi)

            cnt = lax.fori_loop(0, CH // 16, fbody, cnt)

        n_p = cnt
        npadp = ((n_p + GC - 1) // GC) * GC

        def pbody(i, _):
            pos = n_p + i * 16 + i16
            m = pos < npadp
            plsc.store_scatter(cs_src, [pos], zero16, mask=m)
            plsc.store_scatter(cs_meta, [pos], zero16 + DUMP1 * 64, mask=m)
            return 0

        lax.fori_loop(0, GC // 16, pbody, 0)

        def gbody(g, _):
            pltpu.sync_copy(h1augh.at[cs_src.at[pl.ds(g * GC, GC)]], rows)
            for q in range(GC // 16):
                metav = cs_meta[pl.ds(g * GC + q * 16, 16)]
                labv = jnp.bitwise_and(metav, 63)
                glv = plsc.load_gather(rows, [q * 16 + i16, zero16 + D])
                bgv = plsc.load_gather(bg1v, [labv])
                gateb[pl.ds(q * 16, 16)] = _sigmoid16(glv + bgv)
                sidx[pl.ds(q * 16, 16)] = lax.shift_right_logical(metav, 6)

            def ebody(j, _):
                jsp = zero16 + j
                lab_s = jnp.bitwise_and(cs_meta[pl.ds(g * GC + j, 16)][0], 63)
                gsp = jnp.zeros((16,), jnp.float32) + gateb[pl.ds(j, 16)][0]
                for k in range(8):
                    col = k * 16 + i16
                    b1r = b1v[pl.ds(lab_s * 128 + k * 16, 16)]
                    rowk = plsc.load_gather(rows, [jsp, col])
                    plsc.store_scatter(msg, [jsp, col], gsp * (rowk + b1r))
                return 0

            lax.fori_loop(0, GC, ebody, 0)
            pltpu.sync_copy(msg, acc.at[sidx], add=True)
            return 0

        lax.fori_loop(0, npadp // GC, gbody, 0)
        plsc.subcore_barrier()

        for q in range(PS // 16 // GC):
            pltpu.sync_copy(
                acc.at[pl.ds(s * (PS // 16) + q * GC, GC)],
                msg.at[pl.ds(0, GC)])
            pltpu.sync_copy(
                msg.at[pl.ds(0, GC)],
                h1ch.at[c, pl.ds(lo + s * (PS // 16) + q * GC, GC)])
        plsc.subcore_barrier()
        return 0

    lax.fori_loop(0, npass, passbody, 0)


def _layer1_sc(edge_index, edge_attr, map1, n1vec, h1aug, b1, bg1):
    k = pl.kernel(
        _layer1_body,
        out_type=jax.ShapeDtypeStruct((2, N, D), jnp.float32),
        mesh=plsc.VectorSubcoreMesh(**_MESH),
        compiler_params=_SC_PARAMS,
        scratch_types=[
            pltpu.VMEM((N,), jnp.int32),
            pltpu.VMEM((CAP,), jnp.int32),
            pltpu.VMEM((CAP,), jnp.int32),
            pltpu.VMEM((2, CH), jnp.int32),
            pltpu.VMEM((2, CH), jnp.int32),
            pltpu.VMEM((2, CH), jnp.int32),
            pltpu.VMEM((GC, DW), jnp.float32),
            pltpu.VMEM((GC, D), jnp.float32),
            pltpu.VMEM((NUM_LABELS * D,), jnp.float32),
            pltpu.VMEM((64,), jnp.float32),
            pltpu.VMEM((GC + 16,), jnp.float32),
            pltpu.VMEM((GC,), jnp.int32),
            pltpu.VMEM((16,), jnp.int32),
            pltpu.SemaphoreType.DMA,
            pltpu.VMEM_SHARED((ACC1_ROWS, D), jnp.float32),
        ],
    )
    bg1p = jnp.pad(bg1, (0, 64 - NUM_LABELS))
    return k(edge_index, edge_attr, map1, n1vec, h1aug, b1.reshape(-1), bg1p)


def _layer2_body(csh, cmh, cnth, map1h, g2linch, h2hatch, b2h, bg2h,
                 outh,
                 mapv, gval, gsrc, gmeta, rows, msg, b2v, bg2v,
                 gateb, sidx, sidx2, cntb,
                 acc):
    c = lax.axis_index("c")
    s = lax.axis_index("s")
    wid = c * 16 + s
    i16 = lax.iota(jnp.int32, 16)
    zero16 = jnp.zeros((16,), jnp.int32)

    for r in range(RPW2):
        for k in range(8):
            msg[r, pl.ds(k * 16, 16)] = jnp.zeros((16,), jnp.float32)
    pltpu.sync_copy(msg.at[pl.ds(0, RPW2)], acc.at[pl.ds(s * RPW2, RPW2)])

    pltpu.sync_copy(map1h, mapv)
    pltpu.sync_copy(g2linch, gval)
    pltpu.sync_copy(b2h, b2v)
    pltpu.sync_copy(bg2h, bg2v)
    pltpu.sync_copy(cnth.at[wid], cntb)
    n2 = jnp.max(cntb[pl.ds(0, 16)])
    npad = ((n2 + G - 1) // G) * G
    plsc.subcore_barrier()

    def gbody(g, _):
        pltpu.sync_copy(csh.at[pl.ds(wid * CAP + g * G, G)], gsrc)
        pltpu.sync_copy(cmh.at[pl.ds(wid * CAP + g * G, G)],
                        gmeta.at[pl.ds(0, G)])
        for q in range(G // 16):
            srcv = gsrc[pl.ds(q * 16, 16)]
            metav = gmeta[pl.ds(q * 16, 16)]
            slot1 = jnp.maximum(plsc.load_gather(mapv, [srcv]), 0)
            sidx2[pl.ds(q * 16, 16)] = slot1
            labv = jnp.bitwise_and(metav, 63)
            gl = plsc.load_gather(gval, [slot1])
            bg = plsc.load_gather(bg2v, [labv])
            gateb[pl.ds(q * 16, 16)] = _sigmoid16(gl + bg)
            sidx[pl.ds(q * 16, 16)] = lax.shift_right_logical(metav, 6)
        pltpu.sync_copy(h2hatch.at[sidx2], rows)

        def ebody(j, _):
            jsp = zero16 + j
            lab_s = jnp.bitwise_and(gmeta[pl.ds(j, 16)][0], 63)
            gsp = jnp.zeros((16,), jnp.float32) + gateb[pl.ds(j, 16)][0]
            for k in range(8):
                col = k * 16 + i16
                b2r = b2v[pl.ds(lab_s * 128 + k * 16, 16)]
                rowk = plsc.load_gather(rows, [jsp, col])
                plsc.store_scatter(msg, [jsp, col], gsp * (rowk + b2r))
            return 0

        lax.fori_loop(0, G, ebody, 0)
        pltpu.sync_copy(msg, acc.at[sidx], add=True)
        return 0

    lax.fori_loop(0, npad // G, gbody, 0)
    plsc.subcore_barrier()

    pltpu.sync_copy(acc.at[pl.ds(s * RPW2, RPW2)], msg.at[pl.ds(0, RPW2)])
    pltpu.sync_copy(msg.at[pl.ds(0, RPW2)], outh.at[c, pl.ds(s * RPW2, RPW2)])


def _layer2_sc(cs_src, cs_meta, counts, map1, g2lin_c, h2hat_c, b2, bg2):
    k = pl.kernel(
        _layer2_body,
        out_type=jax.ShapeDtypeStruct((2, ACC2_ROWS, D), jnp.float32),
        mesh=plsc.VectorSubcoreMesh(**_MESH),
        compiler_params=_SC_PARAMS,
        scratch_types=[
            pltpu.VMEM((N,), jnp.int32),
            pltpu.VMEM((N,), jnp.float32),
            pltpu.VMEM((G,), jnp.int32),
            pltpu.VMEM((G + 16,), jnp.int32),
            pltpu.VMEM((G, D), jnp.float32),
            pltpu.VMEM((G, D), jnp.float32),
            pltpu.VMEM((NUM_LABELS * D,), jnp.float32),
            pltpu.VMEM((64,), jnp.float32),
            pltpu.VMEM((G + 16,), jnp.float32),
            pltpu.VMEM((G,), jnp.int32),
            pltpu.VMEM((G,), jnp.int32),
            pltpu.VMEM((16,), jnp.int32),
            pltpu.VMEM_SHARED((ACC2_ROWS, D), jnp.float32),
        ],
    )
    bg2p = jnp.pad(bg2, (0, 64 - NUM_LABELS))
    return k(cs_src, cs_meta, counts, map1, g2lin_c, h2hat_c,
             b2.reshape(-1), bg2p)


def kernel(x, embedding, W1, b1, wg1, bg1, W2, b2, wg2, bg2,
           Wi, Wh, bi, bh, edge_index, edge_attr, entA_pos, entB_pos):
    wordA, wordB = _lstm_pool(embedding, Wi, Wh, bi, bh, entA_pos, entB_pos)

    h1aug = jnp.concatenate(
        [x @ W1, (x @ wg1)[:, None],
         jnp.zeros((N, DW - D - 1), jnp.float32)], axis=1)

    idsA = (jnp.arange(B, dtype=jnp.int32)[:, None] * L + entA_pos).reshape(-1)
    idsB = (jnp.arange(B, dtype=jnp.int32)[:, None] * L + entB_pos).reshape(-1)
    idsAB = jnp.concatenate([idsA, idsB])
    map2 = jnp.full((N,), -1, jnp.int32).at[idsAB].set(
        jnp.arange(NSLOT, dtype=jnp.int32))

    cs_src, cs_meta, counts, marks = _edge_scan(edge_index, edge_attr, map2)

    mark = jnp.max(marks, axis=0) > 0
    mi = mark.astype(jnp.int32)
    map1 = jnp.where(mark, jnp.cumsum(mi) - 1, -1).astype(jnp.int32)
    n1vec = jnp.broadcast_to(jnp.sum(mi), (16,)).astype(jnp.int32)

    h1compact = _layer1_sc(edge_index, edge_attr, map1, n1vec, h1aug,
                           b1, bg1)
    h1c = jax.nn.relu(h1compact[0] + h1compact[1])
    h2hat_c = h1c @ W2
    g2lin_c = h1c @ wg2

    acc2 = _layer2_sc(cs_src, cs_meta, counts, map1, g2lin_c, h2hat_c,
                      b2, bg2)
    h2 = jax.nn.relu(acc2[0, :NSLOT] + acc2[1, :NSLOT])

    slotsA = map2[idsA]
    slotsB = map2[idsB]
    nodeA = h2[slotsA].reshape(B, 4, D).mean(axis=1)
    nodeB = h2[slotsB].reshape(B, 4, D).mean(axis=1)

    return (nodeA.reshape(-1), nodeB.reshape(-1),
            wordA.reshape(-1), wordB.reshape(-1))
